# Initial kernel scaffold; baseline (speedup 1.0000x reference)
import jax
import jax.numpy as jnp
from jax.experimental import pallas as pl


def kernel(x, edge_index, emb_table, W1, b1, W2, b2):
    raise NotImplementedError("write your pallas kernel here")


# speedup vs baseline: 23.4305x; 23.4305x over previous
import jax
import jax.numpy as jnp
from jax import lax
from jax.experimental import pallas as pl
from jax.experimental.pallas import tpu as pltpu
from jax.experimental.pallas import tpu_sc as plsc

N = 10000
F = 128
HID = 16
OUT = 7
E = 320000
C = 16
NPAD = 10240
NW = 32
EPT = E // NW
CHUNK = 128
NCHUNK = (EPT + CHUNK - 1) // CHUNK
EPT_PAD = NCHUNK * CHUNK
RPT = NPAD // 16

_mesh = plsc.VectorSubcoreMesh(core_axis_name="c", subcore_axis_name="s")


def _prop_body(y_hbm, row_hbm, col_hbm, zeros_hbm, out_hbm,
               rowv, colv, msg, acc, sem):
    cid = lax.axis_index("c")
    sid = lax.axis_index("s")
    wid = sid * 2 + cid
    pltpu.sync_copy(row_hbm.at[wid], rowv)
    pltpu.sync_copy(col_hbm.at[wid], colv)
    pltpu.sync_copy(zeros_hbm.at[pl.ds(sid * RPT, RPT)],
                    acc.at[pl.ds(sid * RPT, RPT)])
    plsc.subcore_barrier()

    @pl.loop(0, NCHUNK)
    def _chunk(j):
        pltpu.async_copy(y_hbm.at[rowv.at[j]], msg, sem).wait()
        pltpu.sync_copy(msg, acc.at[colv.at[j]], add=True)

    plsc.subcore_barrier()
    pltpu.sync_copy(acc.at[pl.ds(sid * RPT, RPT)],
                    out_hbm.at[cid, pl.ds(sid * RPT, RPT)])


_prop = pl.kernel(
    _prop_body,
    out_type=jax.ShapeDtypeStruct((2, NPAD, C), jnp.float32),
    mesh=_mesh,
    scratch_types=[
        pltpu.VMEM((NCHUNK, CHUNK), jnp.int32),
        pltpu.VMEM((NCHUNK, CHUNK), jnp.int32),
        pltpu.VMEM((CHUNK, C), jnp.float32),
        pltpu.VMEM_SHARED((NPAD, C), jnp.float32),
        pltpu.SemaphoreType.DMA,
    ],
    compiler_params=pltpu.CompilerParams(use_tc_tiling_on_sc=False),
)


def _tc1_body(x_ref, deg16_ref, embfull_ref, w1_ref, w1x_ref,
              y1_ref, dinv_ref):
    x = x_ref[...]
    mean = jnp.sum(x, axis=0, keepdims=True) / N
    sq = jnp.sum(x * x, axis=0, keepdims=True) / N
    var = jnp.maximum(sq - mean * mean, 0.0)
    std = jnp.sqrt(var)
    std = jnp.where(std == 0.0, 1.0, std)
    xn = (x - mean) / std
    c0 = jnp.dot(embfull_ref[...], w1_ref[...],
                 preferred_element_type=jnp.float32)
    h1 = jnp.dot(xn, w1x_ref[...], preferred_element_type=jnp.float32) + c0
    deg = deg16_ref[0] + deg16_ref[1] + 1.0
    dinv = lax.rsqrt(deg)
    dinv_ref[...] = dinv
    y1_ref[...] = dinv * h1


_tc1 = pl.pallas_call(
    _tc1_body,
    out_shape=[
        jax.ShapeDtypeStruct((NPAD, C), jnp.float32),
        jax.ShapeDtypeStruct((NPAD, C), jnp.float32),
    ],
)


def _tc2_body(acc_ref, y1_ref, dinv_ref, b1_ref, w2p_ref, y2_ref):
    dinv = dinv_ref[...]
    y1 = y1_ref[...]
    g1 = dinv * (acc_ref[0] + acc_ref[1] + y1) + b1_ref[...]
    r = jnp.maximum(g1, 0.0)
    h2 = jnp.dot(r, w2p_ref[...], preferred_element_type=jnp.float32)
    y2_ref[...] = dinv * h2


_tc2 = pl.pallas_call(
    _tc2_body,
    out_shape=jax.ShapeDtypeStruct((NPAD, C), jnp.float32),
)


def _tc3_body(acc_ref, y2_ref, dinv_ref, b2_ref, out_ref):
    dinv = dinv_ref[...]
    g2 = dinv * (acc_ref[0] + acc_ref[1] + y2_ref[...]) + b2_ref[...]
    lane = lax.broadcasted_iota(jnp.int32, (NPAD, C), 1)
    g2m = jnp.where(lane < OUT, g2, jnp.float32(-1e30))
    m = jnp.max(g2m, axis=1, keepdims=True)
    ex = jnp.exp(g2m - m)
    lse = jnp.log(jnp.sum(ex, axis=1, keepdims=True))
    out_ref[...] = g2m - m - lse


_tc3 = pl.pallas_call(
    _tc3_body,
    out_shape=jax.ShapeDtypeStruct((NPAD, C), jnp.float32),
)


def kernel(x, edge_index, emb_table, W1, b1, W2, b2):
    f32 = jnp.float32
    x_pad = jnp.pad(x, ((0, NPAD - N), (0, 0)))
    row = edge_index[0].reshape(NW, EPT)
    col = edge_index[1].reshape(NW, EPT)
    row32 = jnp.pad(row, ((0, 0), (0, EPT_PAD - EPT))).reshape(
        NW, NCHUNK, CHUNK)
    col32 = jnp.pad(col, ((0, 0), (0, EPT_PAD - EPT)),
                    constant_values=N).reshape(NW, NCHUNK, CHUNK)
    zeros2d = jnp.zeros((NPAD, C), f32)
    ones2d = jnp.ones((NPAD, C), f32)
    embfull = jnp.concatenate(
        [emb_table, jnp.zeros((F, 1), f32)], axis=1).reshape(1, F * 16)
    w1x = W1.reshape(F, 16, 16)[:, 15, :]
    b1r = b1.reshape(1, HID)
    w2p = jnp.concatenate([W2, jnp.zeros((HID, C - OUT), f32)], axis=1)
    b2r = jnp.concatenate([b2, jnp.zeros((C - OUT,), f32)]).reshape(1, C)

    deg16 = _prop(ones2d, row32, col32, zeros2d)
    y1, dinv = _tc1(x_pad, deg16, embfull, W1, w1x)
    acc1 = _prop(y1, row32, col32, zeros2d)
    y2 = _tc2(acc1, y1, dinv, b1r, w2p)
    acc2 = _prop(y2, row32, col32, zeros2d)
    out = _tc3(acc2, y2, dinv, b2r)
    return out[:N, :OUT]


# speedup vs baseline: 35.3640x; 1.5093x over previous
import jax
import jax.numpy as jnp
from jax import lax
from jax.experimental import pallas as pl
from jax.experimental.pallas import tpu as pltpu
from jax.experimental.pallas import tpu_sc as plsc

N = 10000
F = 128
HID = 16
OUT = 7
E = 320000
C = 16
NPAD = 10240
NW = 32
EPT = E // NW
CHUNK = 128
K = 8
NCHUNK = 80
EPT_PAD = NCHUNK * CHUNK
NSUP = NCHUNK // K
RPT = NPAD // 16

_mesh = plsc.VectorSubcoreMesh(core_axis_name="c", subcore_axis_name="s")


def _stage_and_zero(row_hbm, col_hbm, zeros_hbm, rowv, colv, acc):
    cid = lax.axis_index("c")
    sid = lax.axis_index("s")
    wid = sid * 2 + cid
    pltpu.sync_copy(row_hbm.at[wid], rowv)
    pltpu.sync_copy(col_hbm.at[wid], colv)
    pltpu.sync_copy(zeros_hbm.at[pl.ds(sid * RPT, RPT)],
                    acc.at[pl.ds(sid * RPT, RPT)])
    plsc.subcore_barrier()
    return cid, sid


def _writeback(out_hbm, acc, cid, sid):
    plsc.subcore_barrier()
    pltpu.sync_copy(acc.at[pl.ds(sid * RPT, RPT)],
                    out_hbm.at[cid, pl.ds(sid * RPT, RPT)])


def _prop_body(y_hbm, row_hbm, col_hbm, zeros_hbm, out_hbm,
               rowv, colv, msg, acc, gsem, ssem):
    cid, sid = _stage_and_zero(row_hbm, col_hbm, zeros_hbm, rowv, colv, acc)

    for b in range(K):
        pltpu.async_copy(y_hbm.at[rowv.at[b]], msg.at[b], gsem)

    @pl.loop(0, NSUP)
    def _sup(s):
        p = lax.rem(s, 2)
        base = p * K
        nbase = (1 - p) * K
        for b in range(K):
            pltpu.make_async_copy(
                y_hbm.at[rowv.at[s * K + b]], msg.at[base + b], gsem).wait()

        @pl.when(s >= 1)
        def _drain_prev():
            for b in range(K):
                pltpu.make_async_copy(
                    msg.at[nbase + b],
                    acc.at[colv.at[(s - 1) * K + b]], ssem).wait()

        @pl.when(s < NSUP - 1)
        def _issue_next():
            for b in range(K):
                pltpu.async_copy(
                    y_hbm.at[rowv.at[(s + 1) * K + b]],
                    msg.at[nbase + b], gsem)

        for b in range(K):
            pltpu.async_copy(msg.at[base + b],
                             acc.at[colv.at[s * K + b]], ssem, add=True)

    lastbase = ((NSUP - 1) % 2) * K
    for b in range(K):
        pltpu.make_async_copy(
            msg.at[lastbase + b],
            acc.at[colv.at[(NSUP - 1) * K + b]], ssem).wait()

    _writeback(out_hbm, acc, cid, sid)


_prop = pl.kernel(
    _prop_body,
    out_type=jax.ShapeDtypeStruct((2, NPAD, C), jnp.float32),
    mesh=_mesh,
    scratch_types=[
        pltpu.VMEM((NCHUNK, CHUNK), jnp.int32),
        pltpu.VMEM((NCHUNK, CHUNK), jnp.int32),
        pltpu.VMEM((2 * K, CHUNK, C), jnp.float32),
        pltpu.VMEM_SHARED((NPAD, C), jnp.float32),
        pltpu.SemaphoreType.DMA,
        pltpu.SemaphoreType.DMA,
    ],
    compiler_params=pltpu.CompilerParams(use_tc_tiling_on_sc=False),
)


def _deg_body(ones_hbm, row_hbm, col_hbm, zeros_hbm, out_hbm,
              rowv, colv, onesv, acc, ssem):
    cid, sid = _stage_and_zero(row_hbm, col_hbm, zeros_hbm, rowv, colv, acc)
    pltpu.sync_copy(ones_hbm, onesv)

    @pl.loop(0, NSUP)
    def _sup(s):
        descs = [
            pltpu.async_copy(onesv, acc.at[colv.at[s * K + b]], ssem,
                             add=True)
            for b in range(K)
        ]
        for d in descs:
            d.wait()

    _writeback(out_hbm, acc, cid, sid)


_deg = pl.kernel(
    _deg_body,
    out_type=jax.ShapeDtypeStruct((2, NPAD, C), jnp.float32),
    mesh=_mesh,
    scratch_types=[
        pltpu.VMEM((NCHUNK, CHUNK), jnp.int32),
        pltpu.VMEM((NCHUNK, CHUNK), jnp.int32),
        pltpu.VMEM((CHUNK, C), jnp.float32),
        pltpu.VMEM_SHARED((NPAD, C), jnp.float32),
        pltpu.SemaphoreType.DMA,
    ],
    compiler_params=pltpu.CompilerParams(use_tc_tiling_on_sc=False),
)


def _tc1_body(x_ref, deg16_ref, embfull_ref, w1_ref, w1x_ref,
              y1_ref, dinv_ref):
    x = x_ref[...]
    mean = jnp.sum(x, axis=0, keepdims=True) / N
    sq = jnp.sum(x * x, axis=0, keepdims=True) / N
    var = jnp.maximum(sq - mean * mean, 0.0)
    std = jnp.sqrt(var)
    std = jnp.where(std == 0.0, 1.0, std)
    xn = (x - mean) / std
    c0 = jnp.dot(embfull_ref[...], w1_ref[...],
                 preferred_element_type=jnp.float32)
    h1 = jnp.dot(xn, w1x_ref[...], preferred_element_type=jnp.float32) + c0
    deg = deg16_ref[0] + deg16_ref[1] + 1.0
    dinv = lax.rsqrt(deg)
    dinv_ref[...] = dinv
    y1_ref[...] = dinv * h1


_tc1 = pl.pallas_call(
    _tc1_body,
    out_shape=[
        jax.ShapeDtypeStruct((NPAD, C), jnp.float32),
        jax.ShapeDtypeStruct((NPAD, C), jnp.float32),
    ],
)


def _tc2_body(acc_ref, y1_ref, dinv_ref, b1_ref, w2p_ref, y2_ref):
    dinv = dinv_ref[...]
    y1 = y1_ref[...]
    g1 = dinv * (acc_ref[0] + acc_ref[1] + y1) + b1_ref[...]
    r = jnp.maximum(g1, 0.0)
    h2 = jnp.dot(r, w2p_ref[...], preferred_element_type=jnp.float32)
    y2_ref[...] = dinv * h2


_tc2 = pl.pallas_call(
    _tc2_body,
    out_shape=jax.ShapeDtypeStruct((NPAD, C), jnp.float32),
)


def _tc3_body(acc_ref, y2_ref, dinv_ref, b2_ref, out_ref):
    dinv = dinv_ref[...]
    g2 = dinv * (acc_ref[0] + acc_ref[1] + y2_ref[...]) + b2_ref[...]
    lane = lax.broadcasted_iota(jnp.int32, (NPAD, C), 1)
    g2m = jnp.where(lane < OUT, g2, jnp.float32(-1e30))
    m = jnp.max(g2m, axis=1, keepdims=True)
    ex = jnp.exp(g2m - m)
    lse = jnp.log(jnp.sum(ex, axis=1, keepdims=True))
    out_ref[...] = g2m - m - lse


_tc3 = pl.pallas_call(
    _tc3_body,
    out_shape=jax.ShapeDtypeStruct((NPAD, C), jnp.float32),
)


def kernel(x, edge_index, emb_table, W1, b1, W2, b2):
    f32 = jnp.float32
    x_pad = jnp.pad(x, ((0, NPAD - N), (0, 0)))
    row = edge_index[0].reshape(NW, EPT)
    col = edge_index[1].reshape(NW, EPT)
    row32 = jnp.pad(row, ((0, 0), (0, EPT_PAD - EPT))).reshape(
        NW, NCHUNK, CHUNK)
    col32 = jnp.pad(col, ((0, 0), (0, EPT_PAD - EPT)),
                    constant_values=N).reshape(NW, NCHUNK, CHUNK)
    zeros2d = jnp.zeros((NPAD, C), f32)
    ones_chunk = jnp.ones((CHUNK, C), f32)
    embfull = jnp.concatenate(
        [emb_table, jnp.zeros((F, 1), f32)], axis=1).reshape(1, F * 16)
    w1x = W1.reshape(F, 16, 16)[:, 15, :]
    b1r = b1.reshape(1, HID)
    w2p = jnp.concatenate([W2, jnp.zeros((HID, C - OUT), f32)], axis=1)
    b2r = jnp.concatenate([b2, jnp.zeros((C - OUT,), f32)]).reshape(1, C)

    deg16 = _deg(ones_chunk, row32, col32, zeros2d)
    y1, dinv = _tc1(x_pad, deg16, embfull, W1, w1x)
    acc1 = _prop(y1, row32, col32, zeros2d)
    y2 = _tc2(acc1, y1, dinv, b1r, w2p)
    acc2 = _prop(y2, row32, col32, zeros2d)
    out = _tc3(acc2, y2, dinv, b2r)
    return out[:N, :OUT]


# speedup vs baseline: 36.1189x; 1.0213x over previous
import jax
import jax.numpy as jnp
from jax import lax
from jax.experimental import pallas as pl
from jax.experimental.pallas import tpu as pltpu
from jax.experimental.pallas import tpu_sc as plsc

N = 10000
F = 128
HID = 16
OUT = 7
E = 320000
C = 16
NPAD = 10240
NW = 32
EPT = E // NW
CHUNK = 256
K = 8
NCHUNK = 40
EPT_PAD = NCHUNK * CHUNK
NSUP = NCHUNK // K
RPT = NPAD // 16

_mesh = plsc.VectorSubcoreMesh(core_axis_name="c", subcore_axis_name="s")


def _stage_and_zero(row_hbm, col_hbm, zeros_hbm, rowv, colv, acc):
    cid = lax.axis_index("c")
    sid = lax.axis_index("s")
    wid = sid * 2 + cid
    pltpu.sync_copy(row_hbm.at[wid], rowv)
    pltpu.sync_copy(col_hbm.at[wid], colv)
    pltpu.sync_copy(zeros_hbm.at[pl.ds(sid * RPT, RPT)],
                    acc.at[pl.ds(sid * RPT, RPT)])
    plsc.subcore_barrier()
    return cid, sid


def _writeback(out_hbm, acc, cid, sid):
    plsc.subcore_barrier()
    pltpu.sync_copy(acc.at[pl.ds(sid * RPT, RPT)],
                    out_hbm.at[cid, pl.ds(sid * RPT, RPT)])


def _prop_body(y_hbm, row_hbm, col_hbm, zeros_hbm, out_hbm,
               rowv, colv, msg, acc, gsem, ssem):
    cid, sid = _stage_and_zero(row_hbm, col_hbm, zeros_hbm, rowv, colv, acc)

    for b in range(K):
        pltpu.async_copy(y_hbm.at[rowv.at[b]], msg.at[b], gsem)

    @pl.loop(0, NSUP)
    def _sup(s):
        p = lax.rem(s, 2)
        base = p * K
        nbase = (1 - p) * K
        for b in range(K):
            pltpu.make_async_copy(
                y_hbm.at[rowv.at[s * K + b]], msg.at[base + b], gsem).wait()

        @pl.when(s >= 1)
        def _drain_prev():
            for b in range(K):
                pltpu.make_async_copy(
                    msg.at[nbase + b],
                    acc.at[colv.at[(s - 1) * K + b]], ssem).wait()

        @pl.when(s < NSUP - 1)
        def _issue_next():
            for b in range(K):
                pltpu.async_copy(
                    y_hbm.at[rowv.at[(s + 1) * K + b]],
                    msg.at[nbase + b], gsem)

        for b in range(K):
            pltpu.async_copy(msg.at[base + b],
                             acc.at[colv.at[s * K + b]], ssem, add=True)

    lastbase = ((NSUP - 1) % 2) * K
    for b in range(K):
        pltpu.make_async_copy(
            msg.at[lastbase + b],
            acc.at[colv.at[(NSUP - 1) * K + b]], ssem).wait()

    _writeback(out_hbm, acc, cid, sid)


_prop = pl.kernel(
    _prop_body,
    out_type=jax.ShapeDtypeStruct((2, NPAD, C), jnp.float32),
    mesh=_mesh,
    scratch_types=[
        pltpu.VMEM((NCHUNK, CHUNK), jnp.int32),
        pltpu.VMEM((NCHUNK, CHUNK), jnp.int32),
        pltpu.VMEM((2 * K, CHUNK, C), jnp.float32),
        pltpu.VMEM_SHARED((NPAD, C), jnp.float32),
        pltpu.SemaphoreType.DMA,
        pltpu.SemaphoreType.DMA,
    ],
    compiler_params=pltpu.CompilerParams(use_tc_tiling_on_sc=False),
)


def _deg_body(ones_hbm, row_hbm, col_hbm, zeros_hbm, out_hbm,
              rowv, colv, onesv, acc, ssem):
    cid, sid = _stage_and_zero(row_hbm, col_hbm, zeros_hbm, rowv, colv, acc)
    pltpu.sync_copy(ones_hbm, onesv)

    @pl.loop(0, NSUP)
    def _sup(s):
        descs = [
            pltpu.async_copy(onesv, acc.at[colv.at[s * K + b]], ssem,
                             add=True)
            for b in range(K)
        ]
        for d in descs:
            d.wait()

    _writeback(out_hbm, acc, cid, sid)


_deg = pl.kernel(
    _deg_body,
    out_type=jax.ShapeDtypeStruct((2, NPAD, C), jnp.float32),
    mesh=_mesh,
    scratch_types=[
        pltpu.VMEM((NCHUNK, CHUNK), jnp.int32),
        pltpu.VMEM((NCHUNK, CHUNK), jnp.int32),
        pltpu.VMEM((CHUNK, C), jnp.float32),
        pltpu.VMEM_SHARED((NPAD, C), jnp.float32),
        pltpu.SemaphoreType.DMA,
    ],
    compiler_params=pltpu.CompilerParams(use_tc_tiling_on_sc=False),
)


def _tc1_body(x_ref, deg16_ref, embfull_ref, w1_ref, w1x_ref,
              y1_ref, dinv_ref):
    x = x_ref[...]
    mean = jnp.sum(x, axis=0, keepdims=True) / N
    sq = jnp.sum(x * x, axis=0, keepdims=True) / N
    var = jnp.maximum(sq - mean * mean, 0.0)
    std = jnp.sqrt(var)
    std = jnp.where(std == 0.0, 1.0, std)
    xn = (x - mean) / std
    c0 = jnp.dot(embfull_ref[...], w1_ref[...],
                 preferred_element_type=jnp.float32)
    h1 = jnp.dot(xn, w1x_ref[...], preferred_element_type=jnp.float32) + c0
    deg = deg16_ref[0] + deg16_ref[1] + 1.0
    dinv = lax.rsqrt(deg)
    dinv_ref[...] = dinv
    y1_ref[...] = dinv * h1


_tc1 = pl.pallas_call(
    _tc1_body,
    out_shape=[
        jax.ShapeDtypeStruct((NPAD, C), jnp.float32),
        jax.ShapeDtypeStruct((NPAD, C), jnp.float32),
    ],
)


def _tc2_body(acc_ref, y1_ref, dinv_ref, b1_ref, w2p_ref, y2_ref):
    dinv = dinv_ref[...]
    y1 = y1_ref[...]
    g1 = dinv * (acc_ref[0] + acc_ref[1] + y1) + b1_ref[...]
    r = jnp.maximum(g1, 0.0)
    h2 = jnp.dot(r, w2p_ref[...], preferred_element_type=jnp.float32)
    y2_ref[...] = dinv * h2


_tc2 = pl.pallas_call(
    _tc2_body,
    out_shape=jax.ShapeDtypeStruct((NPAD, C), jnp.float32),
)


def _tc3_body(acc_ref, y2_ref, dinv_ref, b2_ref, out_ref):
    dinv = dinv_ref[...]
    g2 = dinv * (acc_ref[0] + acc_ref[1] + y2_ref[...]) + b2_ref[...]
    lane = lax.broadcasted_iota(jnp.int32, (NPAD, C), 1)
    g2m = jnp.where(lane < OUT, g2, jnp.float32(-1e30))
    m = jnp.max(g2m, axis=1, keepdims=True)
    ex = jnp.exp(g2m - m)
    lse = jnp.log(jnp.sum(ex, axis=1, keepdims=True))
    out_ref[...] = g2m - m - lse


_tc3 = pl.pallas_call(
    _tc3_body,
    out_shape=jax.ShapeDtypeStruct((NPAD, C), jnp.float32),
)


def kernel(x, edge_index, emb_table, W1, b1, W2, b2):
    f32 = jnp.float32
    x_pad = jnp.pad(x, ((0, NPAD - N), (0, 0)))
    row = edge_index[0].reshape(NW, EPT)
    col = edge_index[1].reshape(NW, EPT)
    row32 = jnp.pad(row, ((0, 0), (0, EPT_PAD - EPT))).reshape(
        NW, NCHUNK, CHUNK)
    col32 = jnp.pad(col, ((0, 0), (0, EPT_PAD - EPT)),
                    constant_values=N).reshape(NW, NCHUNK, CHUNK)
    zeros2d = jnp.zeros((NPAD, C), f32)
    ones_chunk = jnp.ones((CHUNK, C), f32)
    embfull = jnp.concatenate(
        [emb_table, jnp.zeros((F, 1), f32)], axis=1).reshape(1, F * 16)
    w1x = W1.reshape(F, 16, 16)[:, 15, :]
    b1r = b1.reshape(1, HID)
    w2p = jnp.concatenate([W2, jnp.zeros((HID, C - OUT), f32)], axis=1)
    b2r = jnp.concatenate([b2, jnp.zeros((C - OUT,), f32)]).reshape(1, C)

    deg16 = _deg(ones_chunk, row32, col32, zeros2d)
    y1, dinv = _tc1(x_pad, deg16, embfull, W1, w1x)
    acc1 = _prop(y1, row32, col32, zeros2d)
    y2 = _tc2(acc1, y1, dinv, b1r, w2p)
    acc2 = _prop(y2, row32, col32, zeros2d)
    out = _tc3(acc2, y2, dinv, b2r)
    return out[:N, :OUT]


# speedup vs baseline: 36.2894x; 1.0047x over previous
import jax
import jax.numpy as jnp
from jax import lax
from jax.experimental import pallas as pl
from jax.experimental.pallas import tpu as pltpu
from jax.experimental.pallas import tpu_sc as plsc

N = 10000
F = 128
HID = 16
OUT = 7
E = 320000
C = 16
NPAD = 10240
NW = 32
EPT = E // NW
CHUNK = 256
K = 8
NCHUNK = 40
EPT_PAD = NCHUNK * CHUNK
NSUP = NCHUNK // K
RPT = NPAD // 16

_mesh = plsc.VectorSubcoreMesh(core_axis_name="c", subcore_axis_name="s")


def _stage_and_zero(row_hbm, col_hbm, zeros_hbm, rowv, colv, acc):
    cid = lax.axis_index("c")
    sid = lax.axis_index("s")
    wid = sid * 2 + cid
    pltpu.sync_copy(row_hbm.at[wid], rowv)
    pltpu.sync_copy(col_hbm.at[wid], colv)
    pltpu.sync_copy(zeros_hbm.at[pl.ds(sid * RPT, RPT)],
                    acc.at[pl.ds(sid * RPT, RPT)])
    plsc.subcore_barrier()
    return cid, sid


def _writeback(out_hbm, acc, cid, sid):
    plsc.subcore_barrier()
    pltpu.sync_copy(acc.at[pl.ds(sid * RPT, RPT)],
                    out_hbm.at[cid, pl.ds(sid * RPT, RPT)])


def _prop_body(y_hbm, row_hbm, col_hbm, zeros_hbm, out_hbm,
               rowv, colv, msg, acc, gsem, ssem):
    cid, sid = _stage_and_zero(row_hbm, col_hbm, zeros_hbm, rowv, colv, acc)

    for b in range(K):
        pltpu.async_copy(y_hbm.at[rowv.at[b]], msg.at[b], gsem)

    @pl.loop(0, NSUP)
    def _sup(s):
        p = lax.rem(s, 2)
        base = p * K
        nbase = (1 - p) * K
        for b in range(K):
            pltpu.make_async_copy(
                y_hbm.at[rowv.at[s * K + b]], msg.at[base + b], gsem).wait()

        @pl.when(s >= 1)
        def _drain_prev():
            for b in range(K):
                pltpu.make_async_copy(
                    msg.at[nbase + b],
                    acc.at[colv.at[(s - 1) * K + b]], ssem).wait()

        @pl.when(s < NSUP - 1)
        def _issue_next():
            for b in range(K):
                pltpu.async_copy(
                    y_hbm.at[rowv.at[(s + 1) * K + b]],
                    msg.at[nbase + b], gsem)

        for b in range(K):
            pltpu.async_copy(msg.at[base + b],
                             acc.at[colv.at[s * K + b]], ssem, add=True)

    lastbase = ((NSUP - 1) % 2) * K
    for b in range(K):
        pltpu.make_async_copy(
            msg.at[lastbase + b],
            acc.at[colv.at[(NSUP - 1) * K + b]], ssem).wait()

    _writeback(out_hbm, acc, cid, sid)


_prop = pl.kernel(
    _prop_body,
    out_type=jax.ShapeDtypeStruct((2, NPAD, C), jnp.float32),
    mesh=_mesh,
    scratch_types=[
        pltpu.VMEM((NCHUNK, CHUNK), jnp.int32),
        pltpu.VMEM((NCHUNK, CHUNK), jnp.int32),
        pltpu.VMEM((2 * K, CHUNK, C), jnp.float32),
        pltpu.VMEM_SHARED((NPAD, C), jnp.float32),
        pltpu.SemaphoreType.DMA,
        pltpu.SemaphoreType.DMA,
    ],
    compiler_params=pltpu.CompilerParams(use_tc_tiling_on_sc=False),
)


def _deg_body(ones_hbm, row_hbm, col_hbm, zeros_hbm, out_hbm,
              rowv, colv, onesv, acc, ssem):
    cid, sid = _stage_and_zero(row_hbm, col_hbm, zeros_hbm, rowv, colv, acc)
    pltpu.sync_copy(ones_hbm, onesv)

    @pl.loop(0, NSUP)
    def _sup(s):
        descs = [
            pltpu.async_copy(onesv, acc.at[colv.at[s * K + b]], ssem,
                             add=True)
            for b in range(K)
        ]
        for d in descs:
            d.wait()

    _writeback(out_hbm, acc, cid, sid)


_deg = pl.kernel(
    _deg_body,
    out_type=jax.ShapeDtypeStruct((2, NPAD, C), jnp.float32),
    mesh=_mesh,
    scratch_types=[
        pltpu.VMEM((NCHUNK, CHUNK), jnp.int32),
        pltpu.VMEM((NCHUNK, CHUNK), jnp.int32),
        pltpu.VMEM((CHUNK, C), jnp.float32),
        pltpu.VMEM_SHARED((NPAD, C), jnp.float32),
        pltpu.SemaphoreType.DMA,
    ],
    compiler_params=pltpu.CompilerParams(use_tc_tiling_on_sc=False),
)


def _tc1a_body(x_ref, embfull_ref, w1_ref, w1x_ref, h1_ref):
    x = x_ref[...]
    mean = jnp.sum(x, axis=0, keepdims=True) / N
    sq = jnp.sum(x * x, axis=0, keepdims=True) / N
    var = jnp.maximum(sq - mean * mean, 0.0)
    std = jnp.sqrt(var)
    std = jnp.where(std == 0.0, 1.0, std)
    xn = (x - mean) / std
    c0 = jnp.dot(embfull_ref[...], w1_ref[...],
                 preferred_element_type=jnp.float32)
    h1_ref[...] = jnp.dot(xn, w1x_ref[...],
                          preferred_element_type=jnp.float32) + c0


_tc1a = pl.pallas_call(
    _tc1a_body,
    out_shape=jax.ShapeDtypeStruct((NPAD, C), jnp.float32),
)


def _tc1b_body(h1_ref, deg16_ref, y1_ref, dinv_ref):
    deg = deg16_ref[0] + deg16_ref[1] + 1.0
    dinv = lax.rsqrt(deg)
    dinv_ref[...] = dinv
    y1_ref[...] = dinv * h1_ref[...]


_tc1b = pl.pallas_call(
    _tc1b_body,
    out_shape=[
        jax.ShapeDtypeStruct((NPAD, C), jnp.float32),
        jax.ShapeDtypeStruct((NPAD, C), jnp.float32),
    ],
)


def _tc2_body(acc_ref, y1_ref, dinv_ref, b1_ref, w2p_ref, y2_ref):
    dinv = dinv_ref[...]
    y1 = y1_ref[...]
    g1 = dinv * (acc_ref[0] + acc_ref[1] + y1) + b1_ref[...]
    r = jnp.maximum(g1, 0.0)
    h2 = jnp.dot(r, w2p_ref[...], preferred_element_type=jnp.float32)
    y2_ref[...] = dinv * h2


_tc2 = pl.pallas_call(
    _tc2_body,
    out_shape=jax.ShapeDtypeStruct((NPAD, C), jnp.float32),
)


def _tc3_body(acc_ref, y2_ref, dinv_ref, b2_ref, out_ref):
    dinv = dinv_ref[...]
    g2 = dinv * (acc_ref[0] + acc_ref[1] + y2_ref[...]) + b2_ref[...]
    lane = lax.broadcasted_iota(jnp.int32, (NPAD, C), 1)
    g2m = jnp.where(lane < OUT, g2, jnp.float32(-1e30))
    m = jnp.max(g2m, axis=1, keepdims=True)
    ex = jnp.exp(g2m - m)
    lse = jnp.log(jnp.sum(ex, axis=1, keepdims=True))
    out_ref[...] = g2m - m - lse


_tc3 = pl.pallas_call(
    _tc3_body,
    out_shape=jax.ShapeDtypeStruct((NPAD, C), jnp.float32),
)


def kernel(x, edge_index, emb_table, W1, b1, W2, b2):
    f32 = jnp.float32
    x_pad = jnp.pad(x, ((0, NPAD - N), (0, 0)))
    row = edge_index[0].reshape(NW, EPT)
    col = edge_index[1].reshape(NW, EPT)
    row32 = jnp.pad(row, ((0, 0), (0, EPT_PAD - EPT))).reshape(
        NW, NCHUNK, CHUNK)
    col32 = jnp.pad(col, ((0, 0), (0, EPT_PAD - EPT)),
                    constant_values=N).reshape(NW, NCHUNK, CHUNK)
    zeros2d = jnp.zeros((NPAD, C), f32)
    ones_chunk = jnp.ones((CHUNK, C), f32)
    embfull = jnp.concatenate(
        [emb_table, jnp.zeros((F, 1), f32)], axis=1).reshape(1, F * 16)
    w1x = W1.reshape(F, 16, 16)[:, 15, :]
    b1r = b1.reshape(1, HID)
    w2p = jnp.concatenate([W2, jnp.zeros((HID, C - OUT), f32)], axis=1)
    b2r = jnp.concatenate([b2, jnp.zeros((C - OUT,), f32)]).reshape(1, C)

    deg16 = _deg(ones_chunk, row32, col32, zeros2d)
    h1 = _tc1a(x_pad, embfull, W1, w1x)
    y1, dinv = _tc1b(h1, deg16)
    acc1 = _prop(y1, row32, col32, zeros2d)
    y2 = _tc2(acc1, y1, dinv, b1r, w2p)
    acc2 = _prop(y2, row32, col32, zeros2d)
    out = _tc3(acc2, y2, dinv, b2r)
    return out[:N, :OUT]


# speedup vs baseline: 36.3402x; 1.0014x over previous
import jax
import jax.numpy as jnp
from jax import lax
from jax.experimental import pallas as pl
from jax.experimental.pallas import tpu as pltpu
from jax.experimental.pallas import tpu_sc as plsc

N = 10000
F = 128
HID = 16
OUT = 7
E = 320000
NPAD = 10240
NW = 32
EPT = E // NW
CHUNK = 256
K = 8
NCHUNK = 40
EPT_PAD = NCHUNK * CHUNK
NSUP = NCHUNK // K
RPT = NPAD // 16

_mesh = plsc.VectorSubcoreMesh(core_axis_name="c", subcore_axis_name="s")
_params = pltpu.CompilerParams(use_tc_tiling_on_sc=False)


def _stage_and_zero(row_hbm, col_hbm, zeros_hbm, rowv, colv, acc):
    cid = lax.axis_index("c")
    sid = lax.axis_index("s")
    wid = sid * 2 + cid
    if rowv is not None:
        pltpu.sync_copy(row_hbm.at[wid], rowv)
    pltpu.sync_copy(col_hbm.at[wid], colv)
    pltpu.sync_copy(zeros_hbm.at[pl.ds(sid * RPT, RPT)],
                    acc.at[pl.ds(sid * RPT, RPT)])
    plsc.subcore_barrier()
    return cid, sid


def _writeback(out_hbm, acc, cid, sid):
    plsc.subcore_barrier()
    pltpu.sync_copy(acc.at[pl.ds(sid * RPT, RPT)],
                    out_hbm.at[cid, pl.ds(sid * RPT, RPT)])


def _make_prop(c):
    def body(y_hbm, row_hbm, col_hbm, zeros_hbm, out_hbm,
             rowv, colv, msg, acc, gsem, ssem):
        cid, sid = _stage_and_zero(row_hbm, col_hbm, zeros_hbm,
                                   rowv, colv, acc)
        for b in range(K):
            pltpu.async_copy(y_hbm.at[rowv.at[b]], msg.at[b], gsem)

        @pl.loop(0, NSUP)
        def _sup(s):
            p = lax.rem(s, 2)
            base = p * K
            nbase = (1 - p) * K
            for b in range(K):
                pltpu.make_async_copy(
                    y_hbm.at[rowv.at[s * K + b]],
                    msg.at[base + b], gsem).wait()

            @pl.when(s >= 1)
            def _drain_prev():
                for b in range(K):
                    pltpu.make_async_copy(
                        msg.at[nbase + b],
                        acc.at[colv.at[(s - 1) * K + b]], ssem).wait()

            @pl.when(s < NSUP - 1)
            def _issue_next():
                for b in range(K):
                    pltpu.async_copy(
                        y_hbm.at[rowv.at[(s + 1) * K + b]],
                        msg.at[nbase + b], gsem)

            for b in range(K):
                pltpu.async_copy(msg.at[base + b],
                                 acc.at[colv.at[s * K + b]], ssem, add=True)

        lastbase = ((NSUP - 1) % 2) * K
        for b in range(K):
            pltpu.make_async_copy(
                msg.at[lastbase + b],
                acc.at[colv.at[(NSUP - 1) * K + b]], ssem).wait()

        _writeback(out_hbm, acc, cid, sid)

    return pl.kernel(
        body,
        out_type=jax.ShapeDtypeStruct((2, NPAD, c), jnp.float32),
        mesh=_mesh,
        scratch_types=[
            pltpu.VMEM((NCHUNK, CHUNK), jnp.int32),
            pltpu.VMEM((NCHUNK, CHUNK), jnp.int32),
            pltpu.VMEM((2 * K, CHUNK, c), jnp.float32),
            pltpu.VMEM_SHARED((NPAD, c), jnp.float32),
            pltpu.SemaphoreType.DMA,
            pltpu.SemaphoreType.DMA,
        ],
        compiler_params=_params,
    )


def _make_deg(c):
    def body(ones_hbm, col_hbm, zeros_hbm, out_hbm,
             colv, onesv, acc, ssem):
        cid, sid = _stage_and_zero(None, col_hbm, zeros_hbm,
                                   None, colv, acc)
        pltpu.sync_copy(ones_hbm, onesv)

        @pl.loop(0, NSUP)
        def _sup(s):
            descs = [
                pltpu.async_copy(onesv, acc.at[colv.at[s * K + b]], ssem,
                                 add=True)
                for b in range(K)
            ]
            for d in descs:
                d.wait()

        _writeback(out_hbm, acc, cid, sid)

    return pl.kernel(
        body,
        out_type=jax.ShapeDtypeStruct((2, NPAD, c), jnp.float32),
        mesh=_mesh,
        scratch_types=[
            pltpu.VMEM((NCHUNK, CHUNK), jnp.int32),
            pltpu.VMEM((CHUNK, c), jnp.float32),
            pltpu.VMEM_SHARED((NPAD, c), jnp.float32),
            pltpu.SemaphoreType.DMA,
        ],
        compiler_params=_params,
    )


_prop16 = _make_prop(16)
_prop8 = _make_prop(8)
_deg1 = _make_deg(1)


def _tc1a_body(x_ref, embfull_ref, w1_ref, w1x_ref, h1_ref):
    x = x_ref[...]
    mean = jnp.sum(x, axis=0, keepdims=True) / N
    sq = jnp.sum(x * x, axis=0, keepdims=True) / N
    var = jnp.maximum(sq - mean * mean, 0.0)
    std = jnp.sqrt(var)
    std = jnp.where(std == 0.0, 1.0, std)
    xn = (x - mean) / std
    c0 = jnp.dot(embfull_ref[...], w1_ref[...],
                 preferred_element_type=jnp.float32)
    h1 = jnp.dot(xn, w1x_ref[...], preferred_element_type=jnp.float32) + c0
    h1_ref[pl.ds(0, N), :] = h1


_tc1a = pl.pallas_call(
    _tc1a_body,
    out_shape=jax.ShapeDtypeStruct((NPAD, HID), jnp.float32),
)


def _tc1b_body(h1_ref, deg1_ref, y1_ref, dinv_ref):
    deg = deg1_ref[0] + deg1_ref[1] + 1.0
    dinv = lax.rsqrt(deg)
    dinv_ref[...] = dinv
    y1_ref[...] = dinv * h1_ref[...]


_tc1b = pl.pallas_call(
    _tc1b_body,
    out_shape=[
        jax.ShapeDtypeStruct((NPAD, HID), jnp.float32),
        jax.ShapeDtypeStruct((NPAD, 1), jnp.float32),
    ],
)


def _tc2_body(acc_ref, y1_ref, dinv_ref, b1_ref, w2p_ref, y2_ref):
    dinv = dinv_ref[...]
    g1 = dinv * (acc_ref[0] + acc_ref[1] + y1_ref[...]) + b1_ref[...]
    r = jnp.maximum(g1, 0.0)
    h2 = jnp.dot(r, w2p_ref[...], preferred_element_type=jnp.float32)
    y2_ref[...] = dinv * h2


_tc2 = pl.pallas_call(
    _tc2_body,
    out_shape=jax.ShapeDtypeStruct((NPAD, 8), jnp.float32),
)


def _tc3_body(acc_ref, y2_ref, dinv_ref, b2_ref, out_ref):
    dinv = dinv_ref[...]
    g2 = dinv * (acc_ref[0] + acc_ref[1] + y2_ref[...]) + b2_ref[...]
    lane = lax.broadcasted_iota(jnp.int32, (NPAD, 8), 1)
    g2m = jnp.where(lane < OUT, g2, jnp.float32(-1e30))
    m = jnp.max(g2m, axis=1, keepdims=True)
    ex = jnp.exp(g2m - m)
    lse = jnp.log(jnp.sum(ex, axis=1, keepdims=True))
    out_ref[...] = g2m - m - lse


_tc3 = pl.pallas_call(
    _tc3_body,
    out_shape=jax.ShapeDtypeStruct((NPAD, 8), jnp.float32),
)


def kernel(x, edge_index, emb_table, W1, b1, W2, b2):
    f32 = jnp.float32
    row = edge_index[0].reshape(NW, EPT)
    col = edge_index[1].reshape(NW, EPT)
    row32 = jnp.pad(row, ((0, 0), (0, EPT_PAD - EPT))).reshape(
        NW, NCHUNK, CHUNK)
    col32 = jnp.pad(col, ((0, 0), (0, EPT_PAD - EPT)),
                    constant_values=N).reshape(NW, NCHUNK, CHUNK)
    zeros16 = jnp.zeros((NPAD, 16), f32)
    zeros8 = jnp.zeros((NPAD, 8), f32)
    zeros1 = jnp.zeros((NPAD, 1), f32)
    ones_chunk = jnp.ones((CHUNK, 1), f32)
    embfull = jnp.concatenate(
        [emb_table, jnp.zeros((F, 1), f32)], axis=1).reshape(1, F * 16)
    w1x = W1.reshape(F, 16, 16)[:, 15, :]
    b1r = b1.reshape(1, HID)
    w2p = jnp.concatenate([W2, jnp.zeros((HID, 1), f32)], axis=1)
    b2r = jnp.concatenate([b2, jnp.zeros((1,), f32)]).reshape(1, 8)

    deg1 = _deg1(ones_chunk, col32, zeros1)
    h1 = _tc1a(x, embfull, W1, w1x)
    y1, dinv = _tc1b(h1, deg1)
    acc1 = _prop16(y1, row32, col32, zeros16)
    y2 = _tc2(acc1, y1, dinv, b1r, w2p)
    acc2 = _prop8(y2, row32, col32, zeros8)
    out = _tc3(acc2, y2, dinv, b2r)
    return out[:N, :OUT]


# speedup vs baseline: 36.6851x; 1.0095x over previous
import jax
import jax.numpy as jnp
from jax import lax
from jax.experimental import pallas as pl
from jax.experimental.pallas import tpu as pltpu
from jax.experimental.pallas import tpu_sc as plsc

N = 10000
F = 128
HID = 16
OUT = 7
E = 320000
NPAD = 10240
NW = 32
EPT = E // NW
CHUNK = 256
K = 8
NCHUNK = 40
EPT_PAD = NCHUNK * CHUNK
NSUP = NCHUNK // K
RPT = NPAD // 16

_mesh = plsc.VectorSubcoreMesh(core_axis_name="c", subcore_axis_name="s")
_params = pltpu.CompilerParams(use_tc_tiling_on_sc=False)


def _stage_and_zero(row_hbm, col_hbm, zeros_hbm, rowv, colv, acc):
    cid = lax.axis_index("c")
    sid = lax.axis_index("s")
    wid = sid * 2 + cid
    if rowv is not None:
        pltpu.sync_copy(row_hbm.at[wid], rowv)
    pltpu.sync_copy(col_hbm.at[wid], colv)
    pltpu.sync_copy(zeros_hbm.at[pl.ds(sid * RPT, RPT)],
                    acc.at[pl.ds(sid * RPT, RPT)])
    plsc.subcore_barrier()
    return cid, sid


def _writeback(out_hbm, acc, cid, sid):
    plsc.subcore_barrier()
    pltpu.sync_copy(acc.at[pl.ds(sid * RPT, RPT)],
                    out_hbm.at[cid, pl.ds(sid * RPT, RPT)])


def _make_prop(c):
    def body(y_hbm, row_hbm, col_hbm, zeros_hbm, out_hbm,
             rowv, colv, msg, acc, gsem, ssem):
        cid, sid = _stage_and_zero(row_hbm, col_hbm, zeros_hbm,
                                   rowv, colv, acc)
        for b in range(K):
            pltpu.async_copy(y_hbm.at[rowv.at[b]], msg.at[b], gsem)

        @pl.loop(0, NSUP)
        def _sup(s):
            p = lax.rem(s, 2)
            base = p * K
            nbase = (1 - p) * K
            for b in range(K):
                pltpu.make_async_copy(
                    y_hbm.at[rowv.at[s * K + b]],
                    msg.at[base + b], gsem).wait()

            @pl.when(s >= 1)
            def _drain_prev():
                for b in range(K):
                    pltpu.make_async_copy(
                        msg.at[nbase + b],
                        acc.at[colv.at[(s - 1) * K + b]], ssem).wait()

            @pl.when(s < NSUP - 1)
            def _issue_next():
                for b in range(K):
                    pltpu.async_copy(
                        y_hbm.at[rowv.at[(s + 1) * K + b]],
                        msg.at[nbase + b], gsem)

            for b in range(K):
                pltpu.async_copy(msg.at[base + b],
                                 acc.at[colv.at[s * K + b]], ssem, add=True)

        lastbase = ((NSUP - 1) % 2) * K
        for b in range(K):
            pltpu.make_async_copy(
                msg.at[lastbase + b],
                acc.at[colv.at[(NSUP - 1) * K + b]], ssem).wait()

        _writeback(out_hbm, acc, cid, sid)

    return pl.kernel(
        body,
        out_type=jax.ShapeDtypeStruct((2, NPAD, c), jnp.float32),
        mesh=_mesh,
        scratch_types=[
            pltpu.VMEM((NCHUNK, CHUNK), jnp.int32),
            pltpu.VMEM((NCHUNK, CHUNK), jnp.int32),
            pltpu.VMEM((2 * K, CHUNK, c), jnp.float32),
            pltpu.VMEM_SHARED((NPAD, c), jnp.float32),
            pltpu.SemaphoreType.DMA,
            pltpu.SemaphoreType.DMA,
        ],
        compiler_params=_params,
    )


def _make_deg(c):
    def body(ones_hbm, col_hbm, zeros_hbm, out_hbm,
             colv, onesv, acc, ssem):
        cid, sid = _stage_and_zero(None, col_hbm, zeros_hbm,
                                   None, colv, acc)
        pltpu.sync_copy(ones_hbm, onesv)

        @pl.loop(0, NSUP)
        def _sup(s):
            descs = [
                pltpu.async_copy(onesv, acc.at[colv.at[s * K + b]], ssem,
                                 add=True)
                for b in range(K)
            ]
            for d in descs:
                d.wait()

        _writeback(out_hbm, acc, cid, sid)

    return pl.kernel(
        body,
        out_type=jax.ShapeDtypeStruct((2, NPAD, c), jnp.float32),
        mesh=_mesh,
        scratch_types=[
            pltpu.VMEM((NCHUNK, CHUNK), jnp.int32),
            pltpu.VMEM((CHUNK, c), jnp.float32),
            pltpu.VMEM_SHARED((NPAD, c), jnp.float32),
            pltpu.SemaphoreType.DMA,
        ],
        compiler_params=_params,
    )


_prop16 = _make_prop(16)
_prop8 = _make_prop(16)
_deg1 = _make_deg(16)


def _tc1a_body(x_ref, embfull_ref, w1_ref, w1x_ref, h1_ref):
    x = x_ref[...]
    mean = jnp.sum(x, axis=0, keepdims=True) / N
    sq = jnp.sum(x * x, axis=0, keepdims=True) / N
    var = jnp.maximum(sq - mean * mean, 0.0)
    std = jnp.sqrt(var)
    std = jnp.where(std == 0.0, 1.0, std)
    xn = (x - mean) / std
    c0 = jnp.dot(embfull_ref[...], w1_ref[...],
                 preferred_element_type=jnp.float32)
    h1 = jnp.dot(xn, w1x_ref[...], preferred_element_type=jnp.float32) + c0
    h1_ref[pl.ds(0, N), :] = h1


_tc1a = pl.pallas_call(
    _tc1a_body,
    out_shape=jax.ShapeDtypeStruct((NPAD, HID), jnp.float32),
)


def _tc1b_body(h1_ref, deg1_ref, y1_ref, dinv_ref):
    deg = deg1_ref[0][:, 0:1] + deg1_ref[1][:, 0:1] + 1.0
    dinv = lax.rsqrt(deg)
    dinv_ref[...] = dinv
    y1_ref[...] = dinv * h1_ref[...]


_tc1b = pl.pallas_call(
    _tc1b_body,
    out_shape=[
        jax.ShapeDtypeStruct((NPAD, HID), jnp.float32),
        jax.ShapeDtypeStruct((NPAD, 1), jnp.float32),
    ],
)


def _tc2_body(acc_ref, y1_ref, dinv_ref, b1_ref, w2p_ref, y2_ref):
    dinv = dinv_ref[...]
    g1 = dinv * (acc_ref[0] + acc_ref[1] + y1_ref[...]) + b1_ref[...]
    r = jnp.maximum(g1, 0.0)
    h2 = jnp.dot(r, w2p_ref[...], preferred_element_type=jnp.float32)
    y2_ref[...] = dinv * h2


_tc2 = pl.pallas_call(
    _tc2_body,
    out_shape=jax.ShapeDtypeStruct((NPAD, 16), jnp.float32),
)


def _tc3_body(acc_ref, y2_ref, dinv_ref, b2_ref, out_ref):
    dinv = dinv_ref[...]
    g2 = dinv * (acc_ref[0] + acc_ref[1] + y2_ref[...]) + b2_ref[...]
    lane = lax.broadcasted_iota(jnp.int32, (NPAD, 16), 1)
    g2m = jnp.where(lane < OUT, g2, jnp.float32(-1e30))
    m = jnp.max(g2m, axis=1, keepdims=True)
    ex = jnp.exp(g2m - m)
    lse = jnp.log(jnp.sum(ex, axis=1, keepdims=True))
    out_ref[...] = g2m - m - lse


_tc3 = pl.pallas_call(
    _tc3_body,
    out_shape=jax.ShapeDtypeStruct((NPAD, 16), jnp.float32),
)


def kernel(x, edge_index, emb_table, W1, b1, W2, b2):
    f32 = jnp.float32
    row = edge_index[0].reshape(NW, EPT)
    col = edge_index[1].reshape(NW, EPT)
    row32 = jnp.pad(row, ((0, 0), (0, EPT_PAD - EPT))).reshape(
        NW, NCHUNK, CHUNK)
    col32 = jnp.pad(col, ((0, 0), (0, EPT_PAD - EPT)),
                    constant_values=N).reshape(NW, NCHUNK, CHUNK)
    zeros16 = jnp.zeros((NPAD, 16), f32)
    zeros1 = jnp.zeros((NPAD, 16), f32)
    ones_chunk = jnp.ones((CHUNK, 16), f32)
    embfull = jnp.concatenate(
        [emb_table, jnp.zeros((F, 1), f32)], axis=1).reshape(1, F * 16)
    w1x = W1.reshape(F, 16, 16)[:, 15, :]
    b1r = b1.reshape(1, HID)
    w2p = jnp.concatenate([W2, jnp.zeros((HID, 16 - OUT), f32)], axis=1)
    b2r = jnp.concatenate([b2, jnp.zeros((16 - OUT,), f32)]).reshape(1, 16)

    deg1 = _deg1(ones_chunk, col32, zeros1)
    h1 = _tc1a(x, embfull, W1, w1x)
    y1, dinv = _tc1b(h1, deg1)
    acc1 = _prop16(y1, row32, col32, zeros16)
    y2 = _tc2(acc1, y1, dinv, b1r, w2p)
    acc2 = _prop8(y2, row32, col32, zeros16)
    out = _tc3(acc2, y2, dinv, b2r)
    return out[:N, :OUT]


# speedup vs baseline: 36.7962x; 1.0030x over previous
import jax
import jax.numpy as jnp
from jax import lax
from jax.experimental import pallas as pl
from jax.experimental.pallas import tpu as pltpu
from jax.experimental.pallas import tpu_sc as plsc

N = 10000
F = 128
HID = 16
OUT = 7
E = 320000
NPAD = 10240
NW = 32
EPT = E // NW
CHUNK = 256
K = 10
NCHUNK = 40
EPT_PAD = NCHUNK * CHUNK
NSUP = NCHUNK // K
RPT = NPAD // 16

_mesh = plsc.VectorSubcoreMesh(core_axis_name="c", subcore_axis_name="s")
_params = pltpu.CompilerParams(use_tc_tiling_on_sc=False)


def _stage_and_zero(row_hbm, col_hbm, zeros_hbm, rowv, colv, acc):
    cid = lax.axis_index("c")
    sid = lax.axis_index("s")
    wid = sid * 2 + cid
    if rowv is not None:
        pltpu.sync_copy(row_hbm.at[wid], rowv)
    pltpu.sync_copy(col_hbm.at[wid], colv)
    pltpu.sync_copy(zeros_hbm.at[pl.ds(sid * RPT, RPT)],
                    acc.at[pl.ds(sid * RPT, RPT)])
    plsc.subcore_barrier()
    return cid, sid


def _writeback(out_hbm, acc, cid, sid):
    plsc.subcore_barrier()
    pltpu.sync_copy(acc.at[pl.ds(sid * RPT, RPT)],
                    out_hbm.at[cid, pl.ds(sid * RPT, RPT)])


def _make_prop(c):
    def body(y_hbm, row_hbm, col_hbm, zeros_hbm, out_hbm,
             rowv, colv, msg, acc, gsem, ssem):
        cid, sid = _stage_and_zero(row_hbm, col_hbm, zeros_hbm,
                                   rowv, colv, acc)
        for b in range(K):
            pltpu.async_copy(y_hbm.at[rowv.at[b]], msg.at[b], gsem)

        @pl.loop(0, NSUP)
        def _sup(s):
            p = lax.rem(s, 2)
            base = p * K
            nbase = (1 - p) * K
            for b in range(K):
                pltpu.make_async_copy(
                    y_hbm.at[rowv.at[s * K + b]],
                    msg.at[base + b], gsem).wait()

            @pl.when(s >= 1)
            def _drain_prev():
                for b in range(K):
                    pltpu.make_async_copy(
                        msg.at[nbase + b],
                        acc.at[colv.at[(s - 1) * K + b]], ssem).wait()

            @pl.when(s < NSUP - 1)
            def _issue_next():
                for b in range(K):
                    pltpu.async_copy(
                        y_hbm.at[rowv.at[(s + 1) * K + b]],
                        msg.at[nbase + b], gsem)

            for b in range(K):
                pltpu.async_copy(msg.at[base + b],
                                 acc.at[colv.at[s * K + b]], ssem, add=True)

        lastbase = ((NSUP - 1) % 2) * K
        for b in range(K):
            pltpu.make_async_copy(
                msg.at[lastbase + b],
                acc.at[colv.at[(NSUP - 1) * K + b]], ssem).wait()

        _writeback(out_hbm, acc, cid, sid)

    return pl.kernel(
        body,
        out_type=jax.ShapeDtypeStruct((2, NPAD, c), jnp.float32),
        mesh=_mesh,
        scratch_types=[
            pltpu.VMEM((NCHUNK, CHUNK), jnp.int32),
            pltpu.VMEM((NCHUNK, CHUNK), jnp.int32),
            pltpu.VMEM((2 * K, CHUNK, c), jnp.float32),
            pltpu.VMEM_SHARED((NPAD, c), jnp.float32),
            pltpu.SemaphoreType.DMA,
            pltpu.SemaphoreType.DMA,
        ],
        compiler_params=_params,
    )


def _make_deg(c):
    def body(ones_hbm, col_hbm, zeros_hbm, out_hbm,
             colv, onesv, acc, ssem):
        cid, sid = _stage_and_zero(None, col_hbm, zeros_hbm,
                                   None, colv, acc)
        pltpu.sync_copy(ones_hbm, onesv)

        @pl.loop(0, NSUP)
        def _sup(s):
            descs = [
                pltpu.async_copy(onesv, acc.at[colv.at[s * K + b]], ssem,
                                 add=True)
                for b in range(K)
            ]
            for d in descs:
                d.wait()

        _writeback(out_hbm, acc, cid, sid)

    return pl.kernel(
        body,
        out_type=jax.ShapeDtypeStruct((2, NPAD, c), jnp.float32),
        mesh=_mesh,
        scratch_types=[
            pltpu.VMEM((NCHUNK, CHUNK), jnp.int32),
            pltpu.VMEM((CHUNK, c), jnp.float32),
            pltpu.VMEM_SHARED((NPAD, c), jnp.float32),
            pltpu.SemaphoreType.DMA,
        ],
        compiler_params=_params,
    )


_prop16 = _make_prop(16)
_prop8 = _make_prop(16)
_deg1 = _make_deg(16)


def _tc1a_body(x_ref, embfull_ref, w1_ref, w1x_ref, h1_ref):
    x = x_ref[...]
    mean = jnp.sum(x, axis=0, keepdims=True) / N
    sq = jnp.sum(x * x, axis=0, keepdims=True) / N
    var = jnp.maximum(sq - mean * mean, 0.0)
    std = jnp.sqrt(var)
    std = jnp.where(std == 0.0, 1.0, std)
    xn = (x - mean) / std
    c0 = jnp.dot(embfull_ref[...], w1_ref[...],
                 preferred_element_type=jnp.float32)
    h1 = jnp.dot(xn, w1x_ref[...], preferred_element_type=jnp.float32) + c0
    h1_ref[pl.ds(0, N), :] = h1


_tc1a = pl.pallas_call(
    _tc1a_body,
    out_shape=jax.ShapeDtypeStruct((NPAD, HID), jnp.float32),
)


def _tc1b_body(h1_ref, deg1_ref, y1_ref, dinv_ref):
    deg = deg1_ref[0][:, 0:1] + deg1_ref[1][:, 0:1] + 1.0
    dinv = lax.rsqrt(deg)
    dinv_ref[...] = dinv
    y1_ref[...] = dinv * h1_ref[...]


_tc1b = pl.pallas_call(
    _tc1b_body,
    out_shape=[
        jax.ShapeDtypeStruct((NPAD, HID), jnp.float32),
        jax.ShapeDtypeStruct((NPAD, 1), jnp.float32),
    ],
)


def _tc2_body(acc_ref, y1_ref, dinv_ref, b1_ref, w2p_ref, y2_ref):
    dinv = dinv_ref[...]
    g1 = dinv * (acc_ref[0] + acc_ref[1] + y1_ref[...]) + b1_ref[...]
    r = jnp.maximum(g1, 0.0)
    h2 = jnp.dot(r, w2p_ref[...], preferred_element_type=jnp.float32)
    y2_ref[...] = dinv * h2


_tc2 = pl.pallas_call(
    _tc2_body,
    out_shape=jax.ShapeDtypeStruct((NPAD, 16), jnp.float32),
)


def _tc3_body(acc_ref, y2_ref, dinv_ref, b2_ref, out_ref):
    dinv = dinv_ref[...]
    g2 = dinv * (acc_ref[0] + acc_ref[1] + y2_ref[...]) + b2_ref[...]
    lane = lax.broadcasted_iota(jnp.int32, (NPAD, 16), 1)
    g2m = jnp.where(lane < OUT, g2, jnp.float32(-1e30))
    m = jnp.max(g2m, axis=1, keepdims=True)
    ex = jnp.exp(g2m - m)
    lse = jnp.log(jnp.sum(ex, axis=1, keepdims=True))
    out_ref[...] = g2m - m - lse


_tc3 = pl.pallas_call(
    _tc3_body,
    out_shape=jax.ShapeDtypeStruct((NPAD, 16), jnp.float32),
)


def kernel(x, edge_index, emb_table, W1, b1, W2, b2):
    f32 = jnp.float32
    row = edge_index[0].reshape(NW, EPT)
    col = edge_index[1].reshape(NW, EPT)
    row32 = jnp.pad(row, ((0, 0), (0, EPT_PAD - EPT))).reshape(
        NW, NCHUNK, CHUNK)
    col32 = jnp.pad(col, ((0, 0), (0, EPT_PAD - EPT)),
                    constant_values=N).reshape(NW, NCHUNK, CHUNK)
    zeros16 = jnp.zeros((NPAD, 16), f32)
    zeros1 = jnp.zeros((NPAD, 16), f32)
    ones_chunk = jnp.ones((CHUNK, 16), f32)
    embfull = jnp.concatenate(
        [emb_table, jnp.zeros((F, 1), f32)], axis=1).reshape(1, F * 16)
    w1x = W1.reshape(F, 16, 16)[:, 15, :]
    b1r = b1.reshape(1, HID)
    w2p = jnp.concatenate([W2, jnp.zeros((HID, 16 - OUT), f32)], axis=1)
    b2r = jnp.concatenate([b2, jnp.zeros((16 - OUT,), f32)]).reshape(1, 16)

    deg1 = _deg1(ones_chunk, col32, zeros1)
    h1 = _tc1a(x, embfull, W1, w1x)
    y1, dinv = _tc1b(h1, deg1)
    acc1 = _prop16(y1, row32, col32, zeros16)
    y2 = _tc2(acc1, y1, dinv, b1r, w2p)
    acc2 = _prop8(y2, row32, col32, zeros16)
    out = _tc3(acc2, y2, dinv, b2r)
    return out[:N, :OUT]
